# baseline (speedup 1.0000x reference)
import functools

import jax
import jax.numpy as jnp
from jax.experimental import pallas as pl
from jax.experimental.pallas import tpu as pltpu
from jax.experimental.pallas import tpu_sc as plsc

DIM = 1024
INTER = 512
NE = 16
TOPK = 2
NG = 18
M = 128
MSH = M.bit_length() - 1
CH = 128
NMETA = 8


def _router_body(x_ref, gw_ref, pos_ref, wts_ref, meta_ref, cum_ref, oh_ref):
    nt = x_ref.shape[0]
    s = jax.lax.dot_general(x_ref[...], gw_ref[...], (((1,), (1,)), ((), ())),
                            preferred_element_type=jnp.float32)
    m = jnp.max(s, axis=1, keepdims=True)
    p = jnp.exp(s - m)
    sm = p / jnp.sum(p, axis=1, keepdims=True)
    lane = jax.lax.broadcasted_iota(jnp.int32, (nt, NE), 1)
    m1 = jnp.max(sm, axis=1, keepdims=True)
    i1 = jnp.min(jnp.where(sm == m1, lane, NE), axis=1, keepdims=True)
    sm2 = jnp.where(lane == i1, -1.0, sm)
    m2 = jnp.max(sm2, axis=1, keepdims=True)
    i2 = jnp.min(jnp.where(sm2 == m2, lane, NE), axis=1, keepdims=True)
    oh_ref[...] = ((lane == i1) | (lane == i2)).astype(jnp.float32)

    r = jax.lax.broadcasted_iota(jnp.int32, (CH, CH), 0)
    c = jax.lax.broadcasted_iota(jnp.int32, (CH, CH), 1)
    tri = (r >= c).astype(jnp.float32)

    def step(i, carry):
        ch = oh_ref[pl.ds(i * CH, CH), :]
        incl = jax.lax.dot_general(tri, ch, (((1,), (0,)), ((), ())),
                                   preferred_element_type=jnp.float32)
        cum_ref[pl.ds(i * CH, CH), :] = incl - ch + carry
        return carry + incl[CH - 1:CH, :]

    counts = jax.lax.fori_loop(0, nt // CH, step, jnp.zeros((1, NE), jnp.float32))

    lane1 = lane[0:1, :]
    off = jnp.zeros((1, NE), jnp.float32)
    for k in range(NE):
        ck = jnp.sum(jnp.where(lane1 == k, counts, 0.0), axis=1, keepdims=True)
        off = off + jnp.where(lane1 > k, ck, 0.0)

    cum = cum_ref[...]
    offb = jnp.broadcast_to(off, (nt, NE))
    pos1 = jnp.sum(jnp.where(lane == i1, cum + offb, 0.0), axis=1, keepdims=True)
    pos2 = jnp.sum(jnp.where(lane == i2, cum + offb, 0.0), axis=1, keepdims=True)
    pos_ref[...] = jnp.concatenate([pos1, pos2], axis=1).astype(jnp.int32)
    wts_ref[...] = jnp.concatenate([m1, m2], axis=1)

    nr = nt * TOPK
    nb_r = nr // M
    nb_x = nt // M
    nb = (nr + nt) // M
    steps = nb_r + (NE - 1) + nb_x
    ngr = NE + 1
    gl = jax.lax.broadcasted_iota(jnp.int32, (1, 32), 1)
    cnt_i = jnp.zeros((1, 32), jnp.int32)
    for k in range(NE):
        ck = jnp.sum(jnp.where(lane1 == k, counts, 0.0), axis=1, keepdims=True)
        cnt_i = cnt_i + jnp.where(gl == k, ck.astype(jnp.int32), 0)
    sizes = jnp.where(gl < NE, cnt_i,
                      jnp.where(gl < ngr, nt, 0))
    offg = jnp.zeros((1, 32), jnp.int32)
    for k in range(ngr):
        ck = jnp.sum(jnp.where(gl == k, sizes, 0), axis=1, keepdims=True)
        offg = offg + jnp.where(gl > k, ck, 0)
    endg = offg + sizes
    fblk = offg >> MSH
    lblk = (endg - 1) >> MSH
    tiles = jnp.where((sizes > 0) & (gl < ngr), lblk - fblk + 1, 0)
    ctiles = jnp.zeros((1, 32), jnp.int32)
    for k in range(ngr):
        ck = jnp.sum(jnp.where(gl == k, tiles, 0), axis=1, keepdims=True)
        ctiles = ctiles + jnp.where(gl >= k, ck, 0)
    sstart = ctiles - tiles

    tcol = jax.lax.broadcasted_iota(jnp.int32, (CH, 1), 0)
    big = jnp.broadcast_to(jnp.where(gl < ngr, ctiles, 10 ** 9), (CH, 32))
    e_arr = jnp.sum((big <= tcol).astype(jnp.int32), axis=1, keepdims=True)

    def glut(v):
        return jnp.sum(jnp.where(
            jax.lax.broadcasted_iota(jnp.int32, (CH, 32), 1) == e_arr,
            jnp.broadcast_to(v, (CH, 32)), 0), axis=1, keepdims=True)

    valid = (e_arr < ngr) & (tcol < steps)
    j = tcol - glut(sstart)
    m_glob = jnp.where(valid, glut(fblk) + j, nb - 1)
    ms = jnp.minimum(m_glob, nb_r - 1)
    mx = jnp.where(valid & (e_arr == NE), m_glob - nb_r,
                   jnp.where(e_arr > NE, nb_x - 1, 0))
    ew = jnp.minimum(e_arr, NE - 1)
    lo = jnp.where(valid, glut(offg), 0)
    hi = jnp.where(valid, glut(endg), 0)
    sh = (e_arr >= NE).astype(jnp.int32)
    fi = (valid & ((j > 0) | ((glut(offg) & (M - 1)) == 0))).astype(jnp.int32)

    mcol = jax.lax.broadcasted_iota(jnp.int32, (CH, NMETA), 1)
    meta = jnp.where(mcol == 0, ms, 0)
    for k, v in enumerate([mx, m_glob, ew, lo, hi, jnp.zeros_like(fi), fi]):
        meta = meta + jnp.where(mcol == k + 1, v, 0)
    meta_ref[...] = jnp.where(mcol == 7, jnp.where(sh == 1, fi + 2, fi), meta)


_SC_NC = 2
_SC_NS = 16
_NW = _SC_NC * _SC_NS


def _dispatch_body(x_hbm, p0_hbm, p1_hbm, a_hbm,
                   xr_a, xr_b, i0_a, i0_b, i1_a, i1_b,
                   stg_a, stg_b, sca_a, sca_b):
    w = jax.lax.axis_index("s") * _SC_NC + jax.lax.axis_index("c")
    nt = x_hbm.shape[0]
    tpw = nt // _NW
    tc = 32
    nch = tpw // tc
    base = w * tpw
    bufs = [(xr_a, i0_a, i1_a, stg_a, sca_a),
            (xr_b, i0_b, i1_b, stg_b, sca_b)]

    def stage(c, b):
        xr, i0, i1, stg, _ = b
        tb = base + c * tc
        return [pltpu.async_copy(x_hbm.at[pl.ds(tb, tc)], xr, stg),
                pltpu.async_copy(p0_hbm.at[pl.ds(tb, tc)], i0, stg),
                pltpu.async_copy(p1_hbm.at[pl.ds(tb, tc)], i1, stg)]

    def scatter(b):
        xr, i0, i1, _, sca = b
        return [pltpu.async_copy(xr, a_hbm.at[i0], sca),
                pltpu.async_copy(xr, a_hbm.at[i1], sca)]

    sc_pend = [[], []]
    st_pend = {0: stage(0, bufs[0])}
    for c in range(nch):
        p = c % 2
        if c + 1 < nch:
            for h in sc_pend[1 - p]:
                h.wait()
            sc_pend[1 - p] = []
            st_pend[c + 1] = stage(c + 1, bufs[1 - p])
        for h in st_pend.pop(c):
            h.wait()
        sc_pend[p] = scatter(bufs[p])
    for lst in sc_pend:
        for h in lst:
            h.wait()


def _gmm_body(meta_ref, a_ref, x_ref, gw_ref, up_ref, dw_ref,
              gws_ref, ups_ref, dws_ref, out_ref):
    t = pl.program_id(0)
    rows = meta_ref[t, 2] * M + jax.lax.broadcasted_iota(jnp.int32, (M, 1), 0)
    inb = (rows >= meta_ref[t, 4]) & (rows < meta_ref[t, 5])
    mask = inb.astype(jnp.float32)
    code = meta_ref[t, 7]

    def ffn(a, gw, up, dw, scale):
        hg = jax.lax.dot_general(a, gw, (((1,), (1,)), ((), ())),
                                 preferred_element_type=jnp.float32)
        hu = jax.lax.dot_general(a, up, (((1,), (1,)), ((), ())),
                                 preferred_element_type=jnp.float32)
        h = hg * jax.lax.logistic(hg) * hu * scale
        return jax.lax.dot_general(h, dw, (((1,), (1,)), ((), ())),
                                   preferred_element_type=jnp.float32)

    def routed_ffn():
        return ffn(a_ref[...], gw_ref[0], up_ref[0], dw_ref[0], mask)

    @pl.when(code == 0)
    def _():
        out_ref[...] += routed_ffn()

    @pl.when(code == 1)
    def _():
        out_ref[...] = routed_ffn()

    def shared_ffn():
        a = x_ref[...]
        return (ffn(a, gws_ref[0], ups_ref[0], dws_ref[0], mask)
                + ffn(a, gws_ref[1], ups_ref[1], dws_ref[1], mask))

    @pl.when(code == 2)
    def _():
        out_ref[...] += shared_ffn()

    @pl.when(code == 3)
    def _():
        out_ref[...] = shared_ffn()


def kernel(x, gate_w, expert_gate_w, expert_up_w, expert_down_w,
           shared_gate_w, shared_up_w, shared_down_w):
    shape = x.shape
    xf = x.reshape(-1, DIM).astype(jnp.float32)
    nt = xf.shape[0]
    nr = nt * TOPK
    rtot = nr + nt
    nb_r = nr // M
    nb_x = nt // M
    steps = nb_r + (NE - 1) + nb_x

    pos, wts, meta = pl.pallas_call(
        _router_body,
        out_shape=(
            jax.ShapeDtypeStruct((nt, TOPK), jnp.int32),
            jax.ShapeDtypeStruct((nt, TOPK), jnp.float32),
            jax.ShapeDtypeStruct((CH, NMETA), jnp.int32),
        ),
        scratch_shapes=[pltpu.VMEM((nt, NE), jnp.float32),
                        pltpu.VMEM((nt, NE), jnp.float32)],
    )(xf, gate_w.astype(jnp.float32))

    pos0 = pos[:, 0]
    pos1 = pos[:, 1]
    dispatch = pl.kernel(
        _dispatch_body,
        out_type=jax.ShapeDtypeStruct((nr, DIM), jnp.float32),
        mesh=plsc.VectorSubcoreMesh(core_axis_name="c", subcore_axis_name="s"),
        scratch_types=(
            [pltpu.VMEM((32, DIM), jnp.float32)] * 2
            + [pltpu.VMEM((32,), jnp.int32)] * 4
            + [pltpu.SemaphoreType.DMA] * 4
        ),
    )
    a_sorted = dispatch(xf, pos0, pos1)

    gws = shared_gate_w.reshape(2, INTER, DIM)
    ups = shared_up_w.reshape(2, INTER, DIM)
    dws = shared_down_w.reshape(DIM, 2, INTER).transpose(1, 0, 2)

    grid_spec = pltpu.PrefetchScalarGridSpec(
        num_scalar_prefetch=1,
        grid=(steps,),
        in_specs=[
            pl.BlockSpec((M, DIM), lambda t, mt: (mt[t, 0], 0)),
            pl.BlockSpec((M, DIM), lambda t, mt: (mt[t, 1], 0)),
            pl.BlockSpec((1, INTER, DIM), lambda t, mt: (mt[t, 3], 0, 0)),
            pl.BlockSpec((1, INTER, DIM), lambda t, mt: (mt[t, 3], 0, 0)),
            pl.BlockSpec((1, DIM, INTER), lambda t, mt: (mt[t, 3], 0, 0)),
            pl.BlockSpec((2, INTER, DIM), lambda t, mt: (0, 0, 0)),
            pl.BlockSpec((2, INTER, DIM), lambda t, mt: (0, 0, 0)),
            pl.BlockSpec((2, DIM, INTER), lambda t, mt: (0, 0, 0)),
        ],
        out_specs=pl.BlockSpec((M, DIM), lambda t, mt: (mt[t, 2], 0)),
    )
    out = pl.pallas_call(
        _gmm_body,
        grid_spec=grid_spec,
        out_shape=jax.ShapeDtypeStruct((rtot, DIM), jnp.float32),
        compiler_params=pltpu.CompilerParams(
            dimension_semantics=("arbitrary",)),
    )(meta, a_sorted, xf,
      expert_gate_w, expert_up_w, expert_down_w, gws, ups, dws)

    y = (wts[:, 0:1] * jnp.take(out, pos0, axis=0)
         + wts[:, 1:2] * jnp.take(out, pos1, axis=0)
         + out[nr:])
    return y.astype(x.dtype).reshape(shape)


# speedup vs baseline: 1.3605x; 1.3605x over previous
import functools

import jax
import jax.numpy as jnp
from jax.experimental import pallas as pl
from jax.experimental.pallas import tpu as pltpu
from jax.experimental.pallas import tpu_sc as plsc

DIM = 1024
INTER = 512
NE = 16
TOPK = 2
NG = 18
M = 256
MSH = M.bit_length() - 1
CH = 128
NMETA = 8


def _router_body(x_ref, gw_ref, pos_ref, wts_ref, meta_ref, cum_ref, oh_ref):
    nt = x_ref.shape[0]
    s = jax.lax.dot_general(x_ref[...], gw_ref[...], (((1,), (1,)), ((), ())),
                            preferred_element_type=jnp.float32)
    m = jnp.max(s, axis=1, keepdims=True)
    p = jnp.exp(s - m)
    sm = p / jnp.sum(p, axis=1, keepdims=True)
    lane = jax.lax.broadcasted_iota(jnp.int32, (nt, NE), 1)
    m1 = jnp.max(sm, axis=1, keepdims=True)
    i1 = jnp.min(jnp.where(sm == m1, lane, NE), axis=1, keepdims=True)
    sm2 = jnp.where(lane == i1, -1.0, sm)
    m2 = jnp.max(sm2, axis=1, keepdims=True)
    i2 = jnp.min(jnp.where(sm2 == m2, lane, NE), axis=1, keepdims=True)
    oh_ref[...] = ((lane == i1) | (lane == i2)).astype(jnp.float32)

    r = jax.lax.broadcasted_iota(jnp.int32, (CH, CH), 0)
    c = jax.lax.broadcasted_iota(jnp.int32, (CH, CH), 1)
    tri = (r >= c).astype(jnp.float32)

    def step(i, carry):
        ch = oh_ref[pl.ds(i * CH, CH), :]
        incl = jax.lax.dot_general(tri, ch, (((1,), (0,)), ((), ())),
                                   preferred_element_type=jnp.float32)
        cum_ref[pl.ds(i * CH, CH), :] = incl - ch + carry
        return carry + incl[CH - 1:CH, :]

    counts = jax.lax.fori_loop(0, nt // CH, step, jnp.zeros((1, NE), jnp.float32))

    lane1 = lane[0:1, :]
    off = jnp.zeros((1, NE), jnp.float32)
    for k in range(NE):
        ck = jnp.sum(jnp.where(lane1 == k, counts, 0.0), axis=1, keepdims=True)
        off = off + jnp.where(lane1 > k, ck, 0.0)

    cum = cum_ref[...]
    offb = jnp.broadcast_to(off, (nt, NE))
    pos1 = jnp.sum(jnp.where(lane == i1, cum + offb, 0.0), axis=1, keepdims=True)
    pos2 = jnp.sum(jnp.where(lane == i2, cum + offb, 0.0), axis=1, keepdims=True)
    pos_ref[...] = jnp.concatenate([pos1, pos2], axis=1).astype(jnp.int32)
    wts_ref[...] = jnp.concatenate([m1, m2], axis=1)

    nr = nt * TOPK
    nb_r = nr // M
    nb_x = nt // M
    nb = (nr + nt) // M
    steps = nb_r + (NE - 1) + nb_x
    ngr = NE + 1
    gl = jax.lax.broadcasted_iota(jnp.int32, (1, 32), 1)
    cnt_i = jnp.zeros((1, 32), jnp.int32)
    for k in range(NE):
        ck = jnp.sum(jnp.where(lane1 == k, counts, 0.0), axis=1, keepdims=True)
        cnt_i = cnt_i + jnp.where(gl == k, ck.astype(jnp.int32), 0)
    sizes = jnp.where(gl < NE, cnt_i,
                      jnp.where(gl < ngr, nt, 0))
    offg = jnp.zeros((1, 32), jnp.int32)
    for k in range(ngr):
        ck = jnp.sum(jnp.where(gl == k, sizes, 0), axis=1, keepdims=True)
        offg = offg + jnp.where(gl > k, ck, 0)
    endg = offg + sizes
    fblk = offg >> MSH
    lblk = (endg - 1) >> MSH
    tiles = jnp.where((sizes > 0) & (gl < ngr), lblk - fblk + 1, 0)
    ctiles = jnp.zeros((1, 32), jnp.int32)
    for k in range(ngr):
        ck = jnp.sum(jnp.where(gl == k, tiles, 0), axis=1, keepdims=True)
        ctiles = ctiles + jnp.where(gl >= k, ck, 0)
    sstart = ctiles - tiles

    tcol = jax.lax.broadcasted_iota(jnp.int32, (CH, 1), 0)
    big = jnp.broadcast_to(jnp.where(gl < ngr, ctiles, 10 ** 9), (CH, 32))
    e_arr = jnp.sum((big <= tcol).astype(jnp.int32), axis=1, keepdims=True)

    def glut(v):
        return jnp.sum(jnp.where(
            jax.lax.broadcasted_iota(jnp.int32, (CH, 32), 1) == e_arr,
            jnp.broadcast_to(v, (CH, 32)), 0), axis=1, keepdims=True)

    valid = (e_arr < ngr) & (tcol < steps)
    j = tcol - glut(sstart)
    m_glob = jnp.where(valid, glut(fblk) + j, nb - 1)
    ms = jnp.minimum(m_glob, nb_r - 1)
    mx = jnp.where(valid & (e_arr == NE), m_glob - nb_r,
                   jnp.where(e_arr > NE, nb_x - 1, 0))
    ew = jnp.minimum(e_arr, NE - 1)
    lo = jnp.where(valid, glut(offg), 0)
    hi = jnp.where(valid, glut(endg), 0)
    sh = (e_arr >= NE).astype(jnp.int32)
    fi = (valid & ((j > 0) | ((glut(offg) & (M - 1)) == 0))).astype(jnp.int32)

    mcol = jax.lax.broadcasted_iota(jnp.int32, (CH, NMETA), 1)
    meta = jnp.where(mcol == 0, ms, 0)
    for k, v in enumerate([mx, m_glob, ew, lo, hi, jnp.zeros_like(fi), fi]):
        meta = meta + jnp.where(mcol == k + 1, v, 0)
    meta_ref[...] = jnp.where(mcol == 7, jnp.where(sh == 1, fi + 2, fi), meta)


_SC_NC = 2
_SC_NS = 16
_NW = _SC_NC * _SC_NS


def _dispatch_body(x_hbm, p0_hbm, p1_hbm, a_hbm,
                   xr_a, xr_b, i0_a, i0_b, i1_a, i1_b,
                   stg_a, stg_b, sca_a, sca_b):
    w = jax.lax.axis_index("s") * _SC_NC + jax.lax.axis_index("c")
    nt = x_hbm.shape[0]
    tpw = nt // _NW
    tc = 32
    nch = tpw // tc
    base = w * tpw
    bufs = [(xr_a, i0_a, i1_a, stg_a, sca_a),
            (xr_b, i0_b, i1_b, stg_b, sca_b)]

    def stage(c, b):
        xr, i0, i1, stg, _ = b
        tb = base + c * tc
        return [pltpu.async_copy(x_hbm.at[pl.ds(tb, tc)], xr, stg),
                pltpu.async_copy(p0_hbm.at[pl.ds(tb, tc)], i0, stg),
                pltpu.async_copy(p1_hbm.at[pl.ds(tb, tc)], i1, stg)]

    def scatter(b):
        xr, i0, i1, _, sca = b
        return [pltpu.async_copy(xr, a_hbm.at[i0], sca),
                pltpu.async_copy(xr, a_hbm.at[i1], sca)]

    sc_pend = [[], []]
    st_pend = {0: stage(0, bufs[0])}
    for c in range(nch):
        p = c % 2
        if c + 1 < nch:
            for h in sc_pend[1 - p]:
                h.wait()
            sc_pend[1 - p] = []
            st_pend[c + 1] = stage(c + 1, bufs[1 - p])
        for h in st_pend.pop(c):
            h.wait()
        sc_pend[p] = scatter(bufs[p])
    for lst in sc_pend:
        for h in lst:
            h.wait()


def _gmm_body(meta_ref, a_ref, x_ref, gw_ref, up_ref, dw_ref,
              gws_ref, ups_ref, dws_ref, out_ref):
    t = pl.program_id(0)
    rows = meta_ref[t, 2] * M + jax.lax.broadcasted_iota(jnp.int32, (M, 1), 0)
    inb = (rows >= meta_ref[t, 4]) & (rows < meta_ref[t, 5])
    mask = inb.astype(jnp.float32)
    code = meta_ref[t, 7]

    def ffn(a, gw, up, dw, scale):
        hg = jax.lax.dot_general(a, gw, (((1,), (1,)), ((), ())),
                                 preferred_element_type=jnp.float32)
        hu = jax.lax.dot_general(a, up, (((1,), (1,)), ((), ())),
                                 preferred_element_type=jnp.float32)
        h = hg * jax.lax.logistic(hg) * hu * scale
        return jax.lax.dot_general(h, dw, (((1,), (1,)), ((), ())),
                                   preferred_element_type=jnp.float32)

    def routed_ffn():
        return ffn(a_ref[...], gw_ref[0], up_ref[0], dw_ref[0], mask)

    @pl.when(code == 0)
    def _():
        out_ref[...] += routed_ffn()

    @pl.when(code == 1)
    def _():
        out_ref[...] = routed_ffn()

    def shared_ffn():
        a = x_ref[...]
        return (ffn(a, gws_ref[0], ups_ref[0], dws_ref[0], mask)
                + ffn(a, gws_ref[1], ups_ref[1], dws_ref[1], mask))

    @pl.when(code == 2)
    def _():
        out_ref[...] += shared_ffn()

    @pl.when(code == 3)
    def _():
        out_ref[...] = shared_ffn()


def kernel(x, gate_w, expert_gate_w, expert_up_w, expert_down_w,
           shared_gate_w, shared_up_w, shared_down_w):
    shape = x.shape
    xf = x.reshape(-1, DIM).astype(jnp.float32)
    nt = xf.shape[0]
    nr = nt * TOPK
    rtot = nr + nt
    nb_r = nr // M
    nb_x = nt // M
    steps = nb_r + (NE - 1) + nb_x

    pos, wts, meta = pl.pallas_call(
        _router_body,
        out_shape=(
            jax.ShapeDtypeStruct((nt, TOPK), jnp.int32),
            jax.ShapeDtypeStruct((nt, TOPK), jnp.float32),
            jax.ShapeDtypeStruct((CH, NMETA), jnp.int32),
        ),
        scratch_shapes=[pltpu.VMEM((nt, NE), jnp.float32),
                        pltpu.VMEM((nt, NE), jnp.float32)],
    )(xf, gate_w.astype(jnp.float32))

    pos0 = pos[:, 0]
    pos1 = pos[:, 1]
    dispatch = pl.kernel(
        _dispatch_body,
        out_type=jax.ShapeDtypeStruct((nr, DIM), jnp.float32),
        mesh=plsc.VectorSubcoreMesh(core_axis_name="c", subcore_axis_name="s"),
        scratch_types=(
            [pltpu.VMEM((32, DIM), jnp.float32)] * 2
            + [pltpu.VMEM((32,), jnp.int32)] * 4
            + [pltpu.SemaphoreType.DMA] * 4
        ),
    )
    a_sorted = dispatch(xf, pos0, pos1)

    gws = shared_gate_w.reshape(2, INTER, DIM)
    ups = shared_up_w.reshape(2, INTER, DIM)
    dws = shared_down_w.reshape(DIM, 2, INTER).transpose(1, 0, 2)

    grid_spec = pltpu.PrefetchScalarGridSpec(
        num_scalar_prefetch=1,
        grid=(steps,),
        in_specs=[
            pl.BlockSpec((M, DIM), lambda t, mt: (mt[t, 0], 0)),
            pl.BlockSpec((M, DIM), lambda t, mt: (mt[t, 1], 0)),
            pl.BlockSpec((1, INTER, DIM), lambda t, mt: (mt[t, 3], 0, 0)),
            pl.BlockSpec((1, INTER, DIM), lambda t, mt: (mt[t, 3], 0, 0)),
            pl.BlockSpec((1, DIM, INTER), lambda t, mt: (mt[t, 3], 0, 0)),
            pl.BlockSpec((2, INTER, DIM), lambda t, mt: (0, 0, 0)),
            pl.BlockSpec((2, INTER, DIM), lambda t, mt: (0, 0, 0)),
            pl.BlockSpec((2, DIM, INTER), lambda t, mt: (0, 0, 0)),
        ],
        out_specs=pl.BlockSpec((M, DIM), lambda t, mt: (mt[t, 2], 0)),
    )
    out = pl.pallas_call(
        _gmm_body,
        grid_spec=grid_spec,
        out_shape=jax.ShapeDtypeStruct((rtot, DIM), jnp.float32),
        compiler_params=pltpu.CompilerParams(
            dimension_semantics=("arbitrary",)),
    )(meta, a_sorted, xf,
      expert_gate_w, expert_up_w, expert_down_w, gws, ups, dws)

    y = (wts[:, 0:1] * jnp.take(out, pos0, axis=0)
         + wts[:, 1:2] * jnp.take(out, pos1, axis=0)
         + out[nr:])
    return y.astype(x.dtype).reshape(shape)


# speedup vs baseline: 1.3695x; 1.0066x over previous
import functools

import jax
import jax.numpy as jnp
from jax.experimental import pallas as pl
from jax.experimental.pallas import tpu as pltpu
from jax.experimental.pallas import tpu_sc as plsc

DIM = 1024
INTER = 512
NE = 16
TOPK = 2
NG = 18
M = 256
MSH = M.bit_length() - 1
CH = 128
CHC = 512
NMETA = 8


def _router_body(x_ref, gw_ref, pos_ref, wts_ref, meta_ref, cum_ref, oh_ref):
    nt = x_ref.shape[0]
    s = jax.lax.dot_general(x_ref[...], gw_ref[...], (((1,), (1,)), ((), ())),
                            preferred_element_type=jnp.float32)
    m = jnp.max(s, axis=1, keepdims=True)
    p = jnp.exp(s - m)
    sm = p / jnp.sum(p, axis=1, keepdims=True)
    lane = jax.lax.broadcasted_iota(jnp.int32, (nt, NE), 1)
    m1 = jnp.max(sm, axis=1, keepdims=True)
    i1 = jnp.min(jnp.where(sm == m1, lane, NE), axis=1, keepdims=True)
    sm2 = jnp.where(lane == i1, -1.0, sm)
    m2 = jnp.max(sm2, axis=1, keepdims=True)
    i2 = jnp.min(jnp.where(sm2 == m2, lane, NE), axis=1, keepdims=True)
    oh_ref[...] = ((lane == i1) | (lane == i2)).astype(jnp.float32)

    r = jax.lax.broadcasted_iota(jnp.int32, (CHC, CHC), 0)
    c = jax.lax.broadcasted_iota(jnp.int32, (CHC, CHC), 1)
    tri = (r >= c).astype(jnp.float32)

    def step(i, carry):
        ch = oh_ref[pl.ds(i * CHC, CHC), :]
        incl = jax.lax.dot_general(tri, ch, (((1,), (0,)), ((), ())),
                                   preferred_element_type=jnp.float32)
        cum_ref[pl.ds(i * CHC, CHC), :] = incl - ch + carry
        return carry + incl[CHC - 1:CHC, :]

    counts = jax.lax.fori_loop(0, nt // CHC, step, jnp.zeros((1, NE), jnp.float32))

    lane1 = lane[0:1, :]
    off = jnp.zeros((1, NE), jnp.float32)
    for k in range(NE):
        ck = jnp.sum(jnp.where(lane1 == k, counts, 0.0), axis=1, keepdims=True)
        off = off + jnp.where(lane1 > k, ck, 0.0)

    cum = cum_ref[...]
    offb = jnp.broadcast_to(off, (nt, NE))
    pos1 = jnp.sum(jnp.where(lane == i1, cum + offb, 0.0), axis=1, keepdims=True)
    pos2 = jnp.sum(jnp.where(lane == i2, cum + offb, 0.0), axis=1, keepdims=True)
    pos_ref[...] = jnp.concatenate([pos1, pos2], axis=1).astype(jnp.int32)
    wts_ref[...] = jnp.concatenate([m1, m2], axis=1)

    nr = nt * TOPK
    nb_r = nr // M
    nb_x = nt // M
    nb = (nr + nt) // M
    steps = nb_r + (NE - 1) + nb_x
    ngr = NE + 1
    gl = jax.lax.broadcasted_iota(jnp.int32, (1, 32), 1)
    cnt_i = jnp.zeros((1, 32), jnp.int32)
    for k in range(NE):
        ck = jnp.sum(jnp.where(lane1 == k, counts, 0.0), axis=1, keepdims=True)
        cnt_i = cnt_i + jnp.where(gl == k, ck.astype(jnp.int32), 0)
    sizes = jnp.where(gl < NE, cnt_i,
                      jnp.where(gl < ngr, nt, 0))
    offg = jnp.zeros((1, 32), jnp.int32)
    for k in range(ngr):
        ck = jnp.sum(jnp.where(gl == k, sizes, 0), axis=1, keepdims=True)
        offg = offg + jnp.where(gl > k, ck, 0)
    endg = offg + sizes
    fblk = offg >> MSH
    lblk = (endg - 1) >> MSH
    tiles = jnp.where((sizes > 0) & (gl < ngr), lblk - fblk + 1, 0)
    ctiles = jnp.zeros((1, 32), jnp.int32)
    for k in range(ngr):
        ck = jnp.sum(jnp.where(gl == k, tiles, 0), axis=1, keepdims=True)
        ctiles = ctiles + jnp.where(gl >= k, ck, 0)
    sstart = ctiles - tiles

    tcol = jax.lax.broadcasted_iota(jnp.int32, (CH, 1), 0)
    big = jnp.broadcast_to(jnp.where(gl < ngr, ctiles, 10 ** 9), (CH, 32))
    e_arr = jnp.sum((big <= tcol).astype(jnp.int32), axis=1, keepdims=True)

    def glut(v):
        return jnp.sum(jnp.where(
            jax.lax.broadcasted_iota(jnp.int32, (CH, 32), 1) == e_arr,
            jnp.broadcast_to(v, (CH, 32)), 0), axis=1, keepdims=True)

    valid = (e_arr < ngr) & (tcol < steps)
    j = tcol - glut(sstart)
    m_glob = jnp.where(valid, glut(fblk) + j, nb - 1)
    ms = jnp.minimum(m_glob, nb_r - 1)
    mx = jnp.where(valid & (e_arr == NE), m_glob - nb_r,
                   jnp.where(e_arr > NE, nb_x - 1, 0))
    ew = jnp.minimum(e_arr, NE - 1)
    lo = jnp.where(valid, glut(offg), 0)
    hi = jnp.where(valid, glut(endg), 0)
    sh = (e_arr >= NE).astype(jnp.int32)
    fi = (valid & ((j > 0) | ((glut(offg) & (M - 1)) == 0))).astype(jnp.int32)

    mcol = jax.lax.broadcasted_iota(jnp.int32, (CH, NMETA), 1)
    meta = jnp.where(mcol == 0, ms, 0)
    for k, v in enumerate([mx, m_glob, ew, lo, hi, jnp.zeros_like(fi), fi]):
        meta = meta + jnp.where(mcol == k + 1, v, 0)
    meta_ref[...] = jnp.where(mcol == 7, jnp.where(sh == 1, fi + 2, fi), meta)


_SC_NC = 2
_SC_NS = 16
_NW = _SC_NC * _SC_NS


def _dispatch_body(x_hbm, p0_hbm, p1_hbm, a_hbm,
                   xr_a, xr_b, i0_a, i0_b, i1_a, i1_b,
                   stg_a, stg_b, sca_a, sca_b):
    w = jax.lax.axis_index("s") * _SC_NC + jax.lax.axis_index("c")
    nt = x_hbm.shape[0]
    tpw = nt // _NW
    tc = 32
    nch = tpw // tc
    base = w * tpw
    bufs = [(xr_a, i0_a, i1_a, stg_a, sca_a),
            (xr_b, i0_b, i1_b, stg_b, sca_b)]

    def stage(c, b):
        xr, i0, i1, stg, _ = b
        tb = base + c * tc
        return [pltpu.async_copy(x_hbm.at[pl.ds(tb, tc)], xr, stg),
                pltpu.async_copy(p0_hbm.at[pl.ds(tb, tc)], i0, stg),
                pltpu.async_copy(p1_hbm.at[pl.ds(tb, tc)], i1, stg)]

    def scatter(b):
        xr, i0, i1, _, sca = b
        return [pltpu.async_copy(xr, a_hbm.at[i0], sca),
                pltpu.async_copy(xr, a_hbm.at[i1], sca)]

    sc_pend = [[], []]
    st_pend = {0: stage(0, bufs[0])}
    for c in range(nch):
        p = c % 2
        if c + 1 < nch:
            for h in sc_pend[1 - p]:
                h.wait()
            sc_pend[1 - p] = []
            st_pend[c + 1] = stage(c + 1, bufs[1 - p])
        for h in st_pend.pop(c):
            h.wait()
        sc_pend[p] = scatter(bufs[p])
    for lst in sc_pend:
        for h in lst:
            h.wait()


def _gmm_body(meta_ref, a_ref, x_ref, gw_ref, up_ref, dw_ref,
              gws_ref, ups_ref, dws_ref, out_ref):
    t = pl.program_id(0)
    rows = meta_ref[t, 2] * M + jax.lax.broadcasted_iota(jnp.int32, (M, 1), 0)
    inb = (rows >= meta_ref[t, 4]) & (rows < meta_ref[t, 5])
    mask = inb.astype(jnp.float32)
    code = meta_ref[t, 7]

    def ffn(a, gw, up, dw, scale):
        hg = jax.lax.dot_general(a, gw, (((1,), (1,)), ((), ())),
                                 preferred_element_type=jnp.float32)
        hu = jax.lax.dot_general(a, up, (((1,), (1,)), ((), ())),
                                 preferred_element_type=jnp.float32)
        h = hg * jax.lax.logistic(hg) * hu * scale
        return jax.lax.dot_general(h, dw, (((1,), (1,)), ((), ())),
                                   preferred_element_type=jnp.float32)

    def routed_ffn():
        return ffn(a_ref[...], gw_ref[0], up_ref[0], dw_ref[0], mask)

    @pl.when(code == 0)
    def _():
        out_ref[...] += routed_ffn()

    @pl.when(code == 1)
    def _():
        out_ref[...] = routed_ffn()

    def shared_ffn():
        a = x_ref[...]
        return (ffn(a, gws_ref[0], ups_ref[0], dws_ref[0], mask)
                + ffn(a, gws_ref[1], ups_ref[1], dws_ref[1], mask))

    @pl.when(code == 2)
    def _():
        out_ref[...] += shared_ffn()

    @pl.when(code == 3)
    def _():
        out_ref[...] = shared_ffn()


def kernel(x, gate_w, expert_gate_w, expert_up_w, expert_down_w,
           shared_gate_w, shared_up_w, shared_down_w):
    shape = x.shape
    xf = x.reshape(-1, DIM).astype(jnp.float32)
    nt = xf.shape[0]
    nr = nt * TOPK
    rtot = nr + nt
    nb_r = nr // M
    nb_x = nt // M
    steps = nb_r + (NE - 1) + nb_x

    pos, wts, meta = pl.pallas_call(
        _router_body,
        out_shape=(
            jax.ShapeDtypeStruct((nt, TOPK), jnp.int32),
            jax.ShapeDtypeStruct((nt, TOPK), jnp.float32),
            jax.ShapeDtypeStruct((CH, NMETA), jnp.int32),
        ),
        scratch_shapes=[pltpu.VMEM((nt, NE), jnp.float32),
                        pltpu.VMEM((nt, NE), jnp.float32)],
    )(xf, gate_w.astype(jnp.float32))

    pos0 = pos[:, 0]
    pos1 = pos[:, 1]
    dispatch = pl.kernel(
        _dispatch_body,
        out_type=jax.ShapeDtypeStruct((nr, DIM), jnp.float32),
        mesh=plsc.VectorSubcoreMesh(core_axis_name="c", subcore_axis_name="s"),
        scratch_types=(
            [pltpu.VMEM((32, DIM), jnp.float32)] * 2
            + [pltpu.VMEM((32,), jnp.int32)] * 4
            + [pltpu.SemaphoreType.DMA] * 4
        ),
    )
    a_sorted = dispatch(xf, pos0, pos1)

    gws = shared_gate_w.reshape(2, INTER, DIM)
    ups = shared_up_w.reshape(2, INTER, DIM)
    dws = shared_down_w.reshape(DIM, 2, INTER).transpose(1, 0, 2)

    grid_spec = pltpu.PrefetchScalarGridSpec(
        num_scalar_prefetch=1,
        grid=(steps,),
        in_specs=[
            pl.BlockSpec((M, DIM), lambda t, mt: (mt[t, 0], 0)),
            pl.BlockSpec((M, DIM), lambda t, mt: (mt[t, 1], 0)),
            pl.BlockSpec((1, INTER, DIM), lambda t, mt: (mt[t, 3], 0, 0)),
            pl.BlockSpec((1, INTER, DIM), lambda t, mt: (mt[t, 3], 0, 0)),
            pl.BlockSpec((1, DIM, INTER), lambda t, mt: (mt[t, 3], 0, 0)),
            pl.BlockSpec((2, INTER, DIM), lambda t, mt: (0, 0, 0)),
            pl.BlockSpec((2, INTER, DIM), lambda t, mt: (0, 0, 0)),
            pl.BlockSpec((2, DIM, INTER), lambda t, mt: (0, 0, 0)),
        ],
        out_specs=pl.BlockSpec((M, DIM), lambda t, mt: (mt[t, 2], 0)),
    )
    out = pl.pallas_call(
        _gmm_body,
        grid_spec=grid_spec,
        out_shape=jax.ShapeDtypeStruct((rtot, DIM), jnp.float32),
        compiler_params=pltpu.CompilerParams(
            dimension_semantics=("arbitrary",)),
    )(meta, a_sorted, xf,
      expert_gate_w, expert_up_w, expert_down_w, gws, ups, dws)

    y = (wts[:, 0:1] * jnp.take(out, pos0, axis=0)
         + wts[:, 1:2] * jnp.take(out, pos1, axis=0)
         + out[nr:])
    return y.astype(x.dtype).reshape(shape)
